# Initial kernel scaffold; baseline (speedup 1.0000x reference)
import jax
import jax.numpy as jnp
from jax.experimental import pallas as pl


def kernel(user_id, adgroup_id, cate_id, is_click, timestamp, is_padding, rel_ad_freqs, user_table, user_w1, user_b1, user_w2, user_b2, ad_adgroup_table, ad_cate_table, ad_w1, ad_b1, ad_w2, ad_b2, action_table, w_ih, w_hh, b_ih, b_hh):
    raise NotImplementedError("write your pallas kernel here")


# speedup vs baseline: 20.2114x; 20.2114x over previous
import functools

import jax
import jax.numpy as jnp
from jax import lax
from jax.experimental import pallas as pl
from jax.experimental.pallas import tpu as pltpu
from jax.experimental.pallas import tpu_sc as plsc

B = 64
L = 50
D = 64
H = 64
N = B * L
NPAD = 3328
NEG_FILL = -1e9


@functools.cache
def _sc_gather4():
  info = plsc.get_sparse_core_info()
  nc, ns = info.num_cores, info.num_subcores
  per_w = NPAD // (nc * ns)
  mesh = plsc.VectorSubcoreMesh(core_axis_name="c", subcore_axis_name="s")
  out = jax.ShapeDtypeStruct((NPAD, D), jnp.float32)

  @functools.partial(
      pl.kernel,
      out_type=(out, out, out, out),
      mesh=mesh,
      scratch_types=[
          pltpu.VMEM((per_w,), jnp.int32),
          pltpu.VMEM((per_w, D), jnp.float32),
          pltpu.SemaphoreType.DMA,
      ],
      compiler_params=pltpu.CompilerParams(use_tc_tiling_on_sc=False),
  )
  def gather4(ag_tbl, ct_tbl, i_ag_lb, i_ct_lb, i_ag_bl, i_ct_bl,
              o_ag_lb, o_ct_lb, o_ag_bl, o_ct_bl, idx_v, rows_v, sem):
    wid = lax.axis_index("s") * nc + lax.axis_index("c")
    base = wid * per_w
    for idx_hbm, tbl, o in ((i_ag_lb, ag_tbl, o_ag_lb),
                            (i_ct_lb, ct_tbl, o_ct_lb),
                            (i_ag_bl, ag_tbl, o_ag_bl),
                            (i_ct_bl, ct_tbl, o_ct_bl)):
      pltpu.sync_copy(idx_hbm.at[pl.ds(base, per_w)], idx_v)
      pltpu.async_copy(tbl.at[idx_v], rows_v, sem).wait()
      pltpu.sync_copy(rows_v, o.at[pl.ds(base, per_w)])

  return gather4


def _tc_body(ag_lb, ct_lb, ag_bl, ct_bl,
             ts_row3, ts_t1, agid_row3, agid_t1, clk_row3, clk_t1, clk_col,
             pad_row3, qp3, posq_t1,
             w1a, w1c, b1, w2, b2, act, wih, whh, bih, bhh,
             out_ref, gi_ref, adbl_ref, pos_ref, all_ref):
  f32 = jnp.float32

  def mlp(a, c):
    h1 = jnp.maximum(
        jnp.dot(a, w1a[...], preferred_element_type=f32)
        + jnp.dot(c, w1c[...], preferred_element_type=f32) + b1[...], 0.0)
    return jnp.dot(h1, w2[...], preferred_element_type=f32) + b2[...]

  ad_lb = mlp(ag_lb[:N, :], ct_lb[:N, :])
  ad_bl = mlp(ag_bl[:N, :], ct_bl[:N, :])
  adbl_ref[...] = ad_bl.reshape(L, B, D)

  tblv = act[...]
  row = lax.broadcasted_iota(jnp.int32, (3, 1), 0)
  tblv = jnp.where(row == 1, 0.0, tblv)
  nrm = jnp.sqrt(jnp.sum(tblv * tblv, axis=1, keepdims=True))
  tblv = tblv * jnp.minimum(1.0, 1.0 / jnp.maximum(nrm, 1e-12))
  clkc = clk_col[...]
  act_emb = (jnp.where(clkc == -1, 1.0, 0.0) * tblv[0:1, :]
             + jnp.where(clkc == 1, 1.0, 0.0) * tblv[2:3, :])

  x = ad_lb + act_emb
  gi = jnp.dot(x, wih[...], preferred_element_type=f32) + bih[...]
  gi_ref[...] = gi.reshape(L, B, 3 * H)

  whh_v = whh[...]
  bhh_v = bhh[...]

  def gru_step(t, h):
    g_i = gi_ref[t]
    gh = jnp.dot(h, whh_v, preferred_element_type=f32) + bhh_v
    r = jax.nn.sigmoid(g_i[:, 0:H] + gh[:, 0:H])
    z = jax.nn.sigmoid(g_i[:, H:2 * H] + gh[:, H:2 * H])
    nn = jnp.tanh(g_i[:, 2 * H:] + r * gh[:, 2 * H:])
    h2 = (1.0 - z) * nn + z * h
    pos_ref[t] = h2
    return h2

  lax.fori_loop(0, L - 1, gru_step, jnp.zeros((B, H), f32))

  pos = pos_ref[...]
  target = ad_lb.reshape(L, B, D)[1:]
  pos_logits = (jnp.sum(pos * target, axis=2)
                - jnp.log(posq_t1[...] + 1e-9))
  all_ref[0] = pos_logits

  ts1 = ts_t1[...]
  ag1 = agid_t1[...]

  def neg_step(l, carry):
    adrow = adbl_ref[l]
    dots = jnp.sum(pos * adrow[None, :, :], axis=2)
    vals = dots - jnp.log(qp3[l] + 1e-9)
    keep = ((pad_row3[l] == 0) & (clk_row3[l] == -1)
            & (ts1 >= ts_row3[l]) & (ag1 != agid_row3[l]))
    all_ref[l + 1] = jnp.where(keep, vals, NEG_FILL)
    return carry

  lax.fori_loop(0, L, neg_step, 0)

  allv = all_ref[...]
  m = jnp.max(allv, axis=0)
  s = jnp.sum(jnp.exp(allv - m[None, :, :]), axis=0)
  per_row = m + jnp.log(s) - pos_logits
  click = clk_t1[...] == 1
  n_click = jnp.sum(jnp.where(click, 1.0, 0.0))
  out_ref[0, 0] = jnp.sum(jnp.where(click, per_row, 0.0)) / n_click


def _tc_call(*args):
  f32 = jnp.float32
  return pl.pallas_call(
      _tc_body,
      out_shape=jax.ShapeDtypeStruct((1, 1), f32),
      out_specs=pl.BlockSpec(memory_space=pltpu.SMEM),
      scratch_shapes=[
          pltpu.VMEM((L, B, 3 * H), f32),
          pltpu.VMEM((L, B, D), f32),
          pltpu.VMEM((L - 1, B, H), f32),
          pltpu.VMEM((L + 1, L - 1, B), f32),
      ],
  )(*args)


def kernel(user_id, adgroup_id, cate_id, is_click, timestamp, is_padding,
           rel_ad_freqs, user_table, user_w1, user_b1, user_w2, user_b2,
           ad_adgroup_table, ad_cate_table, ad_w1, ad_b1, ad_w2, ad_b2,
           action_table, w_ih, w_hh, b_ih, b_hh):
  i32 = jnp.int32
  f32 = jnp.float32

  def pad_idx(a):
    return jnp.pad(a.reshape(-1).astype(i32), (0, NPAD - N))

  i_ag_lb = pad_idx(adgroup_id.T)
  i_ct_lb = pad_idx(cate_id.T)
  i_ag_bl = pad_idx(adgroup_id)
  i_ct_bl = pad_idx(cate_id)

  ag_lb, ct_lb, ag_bl, ct_bl = _sc_gather4()(
      ad_adgroup_table.astype(f32), ad_cate_table.astype(f32),
      i_ag_lb, i_ct_lb, i_ag_bl, i_ct_bl)

  ts_t = timestamp.T.astype(i32)
  ag_t = adgroup_id.T.astype(i32)
  clk_t = is_click.T.astype(i32)
  q = rel_ad_freqs.astype(f32)

  loss2d = _tc_call(
      ag_lb, ct_lb, ag_bl, ct_bl,
      ts_t.reshape(L, 1, B), ts_t[1:],
      ag_t.reshape(L, 1, B), ag_t[1:],
      clk_t.reshape(L, 1, B), clk_t[1:], clk_t.reshape(N, 1),
      is_padding.T.astype(i32).reshape(L, 1, B),
      q.reshape(-1).reshape(L, 1, B),
      q.T[1:],
      ad_w1[:D].astype(f32), ad_w1[D:].astype(f32), ad_b1.reshape(1, -1),
      ad_w2.astype(f32), ad_b2.reshape(1, -1), action_table.astype(f32),
      w_ih.astype(f32), w_hh.astype(f32),
      b_ih.reshape(1, -1), b_hh.reshape(1, -1))
  return loss2d[0, 0]
